# Initial kernel scaffold; baseline (speedup 1.0000x reference)
import functools

import jax
import jax.numpy as jnp
from jax import lax
from jax.experimental import pallas as pl
from jax.experimental.pallas import tpu as pltpu
from jax.experimental.pallas import tpu_sc as plsc

_B = 16384
_D = 128
_NW = 32
_BPW = _B // _NW
_C = 128
_NCHUNK = _BPW // _C
_LANES = 16


def _sc_body(uidx, pidx, uemb, pemb, ubias, pbias,
             partials, ub_out, pb_out,
             idx_u, idx_p, u_buf0, u_buf1, p_buf0, p_buf1,
             ub_v, pb_v, acc_v,
             sem_u0, sem_u1, sem_p0, sem_p1, sem_b):
    wid = lax.axis_index("s") * 2 + lax.axis_index("c")
    base = wid * _BPW

    pltpu.sync_copy(uidx.at[wid], idx_u)
    pltpu.sync_copy(pidx.at[wid], idx_p)

    bias_cps = []
    for c in range(_NCHUNK):
        bias_cps.append(pltpu.async_copy(
            ubias.at[idx_u.at[c]], ub_v.at[pl.ds(c * _C, _C)], sem_b))
        bias_cps.append(pltpu.async_copy(
            pbias.at[idx_p.at[c]], pb_v.at[pl.ds(c * _C, _C)], sem_b))

    u_bufs = (u_buf0, u_buf1)
    p_bufs = (p_buf0, p_buf1)
    sems_u = (sem_u0, sem_u1)
    sems_p = (sem_p0, sem_p1)

    cps = {}
    cps[0] = (pltpu.async_copy(uemb.at[idx_u.at[0]], u_bufs[0], sems_u[0]),
              pltpu.async_copy(pemb.at[idx_p.at[0]], p_bufs[0], sems_p[0]))

    accs = tuple(jnp.zeros((_LANES,), jnp.float32) for _ in range(_D // _LANES))
    for c in range(_NCHUNK):
        s = c % 2
        if c + 1 < _NCHUNK:
            s2 = (c + 1) % 2
            cps[c + 1] = (
                pltpu.async_copy(uemb.at[idx_u.at[c + 1]], u_bufs[s2], sems_u[s2]),
                pltpu.async_copy(pemb.at[idx_p.at[c + 1]], p_bufs[s2], sems_p[s2]),
            )
        cps[c][0].wait()
        cps[c][1].wait()

        ub_ref = u_bufs[s]
        pb_ref = p_bufs[s]

        def row_body(r, a, ub_ref=ub_ref, pb_ref=pb_ref):
            new = []
            for j in range(_D // _LANES):
                uv = ub_ref[r, pl.ds(j * _LANES, _LANES)]
                pv = pb_ref[r, pl.ds(j * _LANES, _LANES)]
                new.append(a[j] + uv * pv)
            return tuple(new)

        accs = lax.fori_loop(0, _C, row_body, accs)

    total = accs[0]
    for j in range(1, _D // _LANES):
        total = total + accs[j]
    acc_v[...] = total

    for cp in bias_cps:
        cp.wait()

    pltpu.sync_copy(acc_v, partials.at[pl.ds(wid * _LANES, _LANES)])
    pltpu.sync_copy(ub_v, ub_out.at[pl.ds(base, _BPW)])
    pltpu.sync_copy(pb_v, pb_out.at[pl.ds(base, _BPW)])


@functools.partial(
    pl.kernel,
    mesh=plsc.VectorSubcoreMesh(core_axis_name="c", subcore_axis_name="s"),
    out_type=[
        jax.ShapeDtypeStruct((_NW * _LANES,), jnp.float32),
        jax.ShapeDtypeStruct((_B, 1), jnp.float32),
        jax.ShapeDtypeStruct((_B, 1), jnp.float32),
    ],
    scratch_types=[
        pltpu.VMEM((_NCHUNK, _C), jnp.int32),
        pltpu.VMEM((_NCHUNK, _C), jnp.int32),
        pltpu.VMEM((_C, _D), jnp.float32),
        pltpu.VMEM((_C, _D), jnp.float32),
        pltpu.VMEM((_C, _D), jnp.float32),
        pltpu.VMEM((_C, _D), jnp.float32),
        pltpu.VMEM((_BPW, 1), jnp.float32),
        pltpu.VMEM((_BPW, 1), jnp.float32),
        pltpu.VMEM((_LANES,), jnp.float32),
        pltpu.SemaphoreType.DMA,
        pltpu.SemaphoreType.DMA,
        pltpu.SemaphoreType.DMA,
        pltpu.SemaphoreType.DMA,
        pltpu.SemaphoreType.DMA,
    ],
)
def _sc_gather_dot(uidx, pidx, uemb, pemb, ubias, pbias, *rest):
    _sc_body(uidx, pidx, uemb, pemb, ubias, pbias, *rest)


def _tc_finish(partials_ref, ub_ref, pb_ref, out_ref):
    s = jnp.sum(partials_ref[...])
    out_ref[...] = jax.nn.sigmoid(ub_ref[...] + pb_ref[...] + s)


def kernel(inputs, user_embedding, user_bias, product_embedding, product_bias):
    u_idx = inputs[:, 0].astype(jnp.int32).reshape(_NW, _NCHUNK, _C)
    p_idx = inputs[:, 1].astype(jnp.int32).reshape(_NW, _NCHUNK, _C)

    partials, ub, pb = _sc_gather_dot(
        u_idx, p_idx, user_embedding, product_embedding,
        user_bias, product_bias)

    out = pl.pallas_call(
        _tc_finish,
        out_shape=jax.ShapeDtypeStruct((_B // _D, _D), jnp.float32),
    )(partials.reshape(4, _D), ub.reshape(_B // _D, _D), pb.reshape(_B // _D, _D))
    return out.reshape(_B, 1)


# speedup vs baseline: 1.6973x; 1.6973x over previous
import functools

import jax
import jax.numpy as jnp
from jax import lax
from jax.experimental import pallas as pl
from jax.experimental.pallas import tpu as pltpu
from jax.experimental.pallas import tpu_sc as plsc

_B = 16384
_D = 128
_NW = 32
_BPW = _B // _NW
_C = 128
_NCHUNK = _BPW // _C
_LANES = 16


def _sc_body(uidx, pidx, uemb, pemb, ubias, pbias,
             partials, ub_out, pb_out,
             idx_u, idx_p, u_buf0, u_buf1, p_buf0, p_buf1,
             ub_v, pb_v, acc_v,
             sem_u0, sem_u1, sem_p0, sem_p1, sem_b):
    wid = lax.axis_index("s") * 2 + lax.axis_index("c")
    base = wid * _BPW

    pltpu.sync_copy(uidx.at[wid], idx_u)
    pltpu.sync_copy(pidx.at[wid], idx_p)

    bias_cps = []
    for c in range(_NCHUNK):
        bias_cps.append(pltpu.async_copy(
            ubias.at[idx_u.at[c]], ub_v.at[pl.ds(c * _C, _C)], sem_b))
        bias_cps.append(pltpu.async_copy(
            pbias.at[idx_p.at[c]], pb_v.at[pl.ds(c * _C, _C)], sem_b))

    u_bufs = (u_buf0, u_buf1)
    p_bufs = (p_buf0, p_buf1)
    sems_u = (sem_u0, sem_u1)
    sems_p = (sem_p0, sem_p1)

    cps = {}
    cps[0] = (pltpu.async_copy(uemb.at[idx_u.at[0]], u_bufs[0], sems_u[0]),
              pltpu.async_copy(pemb.at[idx_p.at[0]], p_bufs[0], sems_p[0]))

    accs = tuple(jnp.zeros((_LANES,), jnp.float32) for _ in range(_D // _LANES))
    for c in range(_NCHUNK):
        s = c % 2
        if c + 1 < _NCHUNK:
            s2 = (c + 1) % 2
            cps[c + 1] = (
                pltpu.async_copy(uemb.at[idx_u.at[c + 1]], u_bufs[s2], sems_u[s2]),
                pltpu.async_copy(pemb.at[idx_p.at[c + 1]], p_bufs[s2], sems_p[s2]),
            )
        cps[c][0].wait()
        cps[c][1].wait()

        ub_ref = u_bufs[s]
        pb_ref = p_bufs[s]

        def row_body(r, a, ub_ref=ub_ref, pb_ref=pb_ref):
            new = []
            for j in range(_D // _LANES):
                uv = ub_ref[r, pl.ds(j * _LANES, _LANES)]
                pv = pb_ref[r, pl.ds(j * _LANES, _LANES)]
                new.append(a[j] + uv * pv)
            return tuple(new)

        accs = lax.fori_loop(0, _C, row_body, accs)

    total = accs[0]
    for j in range(1, _D // _LANES):
        total = total + accs[j]
    acc_v[...] = total

    for cp in bias_cps:
        cp.wait()

    pltpu.sync_copy(acc_v, partials.at[pl.ds(wid * _LANES, _LANES)])
    pltpu.sync_copy(ub_v, ub_out.at[pl.ds(base, _BPW)])
    pltpu.sync_copy(pb_v, pb_out.at[pl.ds(base, _BPW)])


@functools.partial(
    pl.kernel,
    mesh=plsc.VectorSubcoreMesh(core_axis_name="c", subcore_axis_name="s"),
    out_type=[
        jax.ShapeDtypeStruct((_NW * _LANES,), jnp.float32),
        jax.ShapeDtypeStruct((_B,), jnp.float32),
        jax.ShapeDtypeStruct((_B,), jnp.float32),
    ],
    scratch_types=[
        pltpu.VMEM((_NCHUNK, _C), jnp.int32),
        pltpu.VMEM((_NCHUNK, _C), jnp.int32),
        pltpu.VMEM((_C, _D), jnp.float32),
        pltpu.VMEM((_C, _D), jnp.float32),
        pltpu.VMEM((_C, _D), jnp.float32),
        pltpu.VMEM((_C, _D), jnp.float32),
        pltpu.VMEM((_BPW,), jnp.float32),
        pltpu.VMEM((_BPW,), jnp.float32),
        pltpu.VMEM((_LANES,), jnp.float32),
        pltpu.SemaphoreType.DMA,
        pltpu.SemaphoreType.DMA,
        pltpu.SemaphoreType.DMA,
        pltpu.SemaphoreType.DMA,
        pltpu.SemaphoreType.DMA,
    ],
)
def _sc_gather_dot(uidx, pidx, uemb, pemb, ubias, pbias, *rest):
    _sc_body(uidx, pidx, uemb, pemb, ubias, pbias, *rest)


def _tc_finish(partials_ref, ub_ref, pb_ref, out_ref):
    s = jnp.sum(partials_ref[...])
    out_ref[...] = jax.nn.sigmoid(ub_ref[...] + pb_ref[...] + s)


def kernel(inputs, user_embedding, user_bias, product_embedding, product_bias):
    u_idx = inputs[:, 0].astype(jnp.int32).reshape(_NW, _NCHUNK, _C)
    p_idx = inputs[:, 1].astype(jnp.int32).reshape(_NW, _NCHUNK, _C)

    partials, ub, pb = _sc_gather_dot(
        u_idx, p_idx, user_embedding, product_embedding,
        user_bias.reshape(-1), product_bias.reshape(-1))

    out = pl.pallas_call(
        _tc_finish,
        out_shape=jax.ShapeDtypeStruct((_B // _D, _D), jnp.float32),
    )(partials.reshape(4, _D), ub.reshape(_B // _D, _D), pb.reshape(_B // _D, _D))
    return out.reshape(_B, 1)
